# baseline (speedup 1.0000x reference)
import functools

import jax
import jax.numpy as jnp
from jax import lax
from jax.experimental import pallas as pl
from jax.experimental.pallas import tpu as pltpu
from jax.experimental.pallas import tpu_sc as plsc

B = 16384
W_DIM, C_DIM, SC_DIM, I_DIM = 64, 16, 32, 16
N_W, N_P, N_C = 16, 16, 8
EMB = W_DIM + C_DIM + SC_DIM + I_DIM
FC_IN = EMB + N_W + N_P + N_C
FC1, FC2 = 128, 32
NUM_WORKERS, NUM_CATS, NUM_SUBCATS, NUM_INDS = 1000000, 1000, 100000, 1000

R_BIG = 16
R_SMALL = 8
OFF_W, OFF_C, OFF_S, OFF_I = 0, W_DIM, W_DIM + C_DIM, W_DIM + C_DIM + SC_DIM


def _make_sc_gather():
    info = plsc.get_sparse_core_info()
    nw = info.num_cores * info.num_subcores
    b_per_w = B // nw
    mesh = plsc.VectorSubcoreMesh(core_axis_name="c", subcore_axis_name="s")

    @functools.partial(
        pl.kernel,
        mesh=mesh,
        out_type=jax.ShapeDtypeStruct((B, EMB), jnp.float32),
        scratch_types=[
            pltpu.VMEM((b_per_w,), jnp.int32),
            pltpu.VMEM((b_per_w,), jnp.int32),
            pltpu.VMEM((b_per_w,), jnp.int32),
            pltpu.VMEM((b_per_w,), jnp.int32),
            pltpu.VMEM((b_per_w, EMB), jnp.float32),
            pltpu.SemaphoreType.DMA,
        ],
    )
    def sc_gather(wid_hbm, cid_hbm, sid_hbm, iid_hbm,
                  ww_hbm, wc_hbm, ws_hbm, wi_hbm,
                  out_hbm,
                  wi_v, ci_v, si_v, ii_v, ob_v, sem):
        w = lax.axis_index("s") * info.num_cores + lax.axis_index("c")
        base = w * b_per_w
        pltpu.sync_copy(wid_hbm.at[pl.ds(base, b_per_w)], wi_v)
        pltpu.sync_copy(cid_hbm.at[pl.ds(base, b_per_w)], ci_v)
        pltpu.sync_copy(sid_hbm.at[pl.ds(base, b_per_w)], si_v)
        pltpu.sync_copy(iid_hbm.at[pl.ds(base, b_per_w)], ii_v)

        @pl.loop(0, b_per_w // 16)
        def _grp(g):
            gbase = g * 16
            wv = wi_v[pl.ds(gbase, 16)]
            cv = ci_v[pl.ds(gbase, 16)]
            sv = si_v[pl.ds(gbase, 16)]
            iv = ii_v[pl.ds(gbase, 16)]
            for j in range(16):
                i = gbase + j
                wid, cid, sid, iid = wv[j], cv[j], sv[j], iv[j]
                pltpu.async_copy(ww_hbm.at[wid],
                                 ob_v.at[i, pl.ds(OFF_W, W_DIM)], sem)
                pltpu.async_copy(wc_hbm.at[cid],
                                 ob_v.at[i, pl.ds(OFF_C, C_DIM)], sem)
                pltpu.async_copy(ws_hbm.at[sid],
                                 ob_v.at[i, pl.ds(OFF_S, SC_DIM)], sem)
                pltpu.async_copy(wi_hbm.at[iid],
                                 ob_v.at[i, pl.ds(OFF_I, I_DIM)], sem)

        pltpu.make_async_copy(
            out_hbm.at[pl.ds(0, b_per_w)], ob_v, sem).wait()
        pltpu.sync_copy(ob_v, out_hbm.at[pl.ds(base, b_per_w)])

    return sc_gather


_sc_gather = None


def _mlp_body(emb_ref, nw_ref, np_ref, nc_ref,
              w1_ref, b1_ref, w2_ref, b2_ref, w3t_ref, b3_ref, out_ref):
    feats = jnp.concatenate(
        [emb_ref[...], nw_ref[...], np_ref[...], nc_ref[...]], axis=1)
    x = jnp.dot(feats, w1_ref[...], preferred_element_type=jnp.float32)
    x = jnp.maximum(x + b1_ref[...], 0.0)
    x = jnp.dot(x, w2_ref[...], preferred_element_type=jnp.float32)
    x = jnp.maximum(x + b2_ref[...], 0.0)
    out_ref[...] = jnp.sum(x * w3t_ref[...], axis=1, keepdims=True) + b3_ref[...]


def _mlp(emb, nw, npf, ncf, w1, b1, w2, b2, w3, b3, bt=2048):
    grid = B // bt
    ds = lambda i: (i, 0)
    ws = lambda i: (0, 0)
    return pl.pallas_call(
        _mlp_body,
        grid=(grid,),
        in_specs=[
            pl.BlockSpec((bt, EMB), ds),
            pl.BlockSpec((bt, N_W), ds),
            pl.BlockSpec((bt, N_P), ds),
            pl.BlockSpec((bt, N_C), ds),
            pl.BlockSpec((FC_IN, FC1), ws),
            pl.BlockSpec((1, FC1), ws),
            pl.BlockSpec((FC1, FC2), ws),
            pl.BlockSpec((1, FC2), ws),
            pl.BlockSpec((1, FC2), ws),
            pl.BlockSpec((1, 1), ws),
        ],
        out_specs=pl.BlockSpec((bt, 1), ds),
        out_shape=jax.ShapeDtypeStruct((B, 1), jnp.float32),
    )(emb, nw, npf, ncf,
      w1, b1.reshape(1, FC1), w2, b2.reshape(1, FC2),
      w3.reshape(1, FC2), b3.reshape(1, 1))


def kernel(worker_ids, cat_ids, sub_cat_ids, ind_ids,
           numeric_worker_feats, numeric_project_feats, numeric_context_feats,
           W_worker, W_cat, W_sub, W_ind, W1, b1, W2, b2, W3, b3):
    global _sc_gather
    if _sc_gather is None:
        _sc_gather = _make_sc_gather()
    emb = _sc_gather(
        worker_ids.astype(jnp.int32), cat_ids.astype(jnp.int32),
        sub_cat_ids.astype(jnp.int32), ind_ids.astype(jnp.int32),
        W_worker, W_cat, W_sub, W_ind)
    return _mlp(emb, numeric_worker_feats, numeric_project_feats,
                numeric_context_feats, W1, b1, W2, b2, W3, b3)


# speedup vs baseline: 1.4620x; 1.4620x over previous
import functools

import jax
import jax.numpy as jnp
from jax import lax
from jax.experimental import pallas as pl
from jax.experimental.pallas import tpu as pltpu
from jax.experimental.pallas import tpu_sc as plsc

B = 16384
W_DIM, C_DIM, SC_DIM, I_DIM = 64, 16, 32, 16
N_W, N_P, N_C = 16, 16, 8
EMB = W_DIM + C_DIM + SC_DIM + I_DIM
FC_IN = EMB + N_W + N_P + N_C
FC1, FC2 = 128, 32
NUM_WORKERS, NUM_CATS, NUM_SUBCATS, NUM_INDS = 1000000, 1000, 100000, 1000

OFF_W, OFF_C, OFF_S, OFF_I = 0, W_DIM, W_DIM + C_DIM, W_DIM + C_DIM + SC_DIM


def _make_sc_gather():
    info = plsc.get_sparse_core_info()
    nw = info.num_cores * info.num_subcores
    b_per_w = B // nw
    mesh = plsc.VectorSubcoreMesh(core_axis_name="c", subcore_axis_name="s")

    @functools.partial(
        pl.kernel,
        mesh=mesh,
        compiler_params=pltpu.CompilerParams(needs_layout_passes=False),
        out_type=jax.ShapeDtypeStruct((B, EMB), jnp.float32),
        scratch_types=[
            pltpu.VMEM((b_per_w,), jnp.int32),
            pltpu.VMEM((b_per_w,), jnp.int32),
            pltpu.VMEM((b_per_w,), jnp.int32),
            pltpu.VMEM((b_per_w,), jnp.int32),
            pltpu.VMEM((C_DIM, NUM_CATS), jnp.float32),
            pltpu.VMEM((I_DIM, NUM_INDS), jnp.float32),
            pltpu.VMEM((b_per_w, EMB), jnp.float32),
            pltpu.SemaphoreType.DMA,
        ],
    )
    def sc_gather(wid_hbm, cid_hbm, sid_hbm, iid_hbm,
                  ww_hbm, wct_hbm, ws_hbm, wit_hbm,
                  out_hbm,
                  wi_v, ci_v, si_v, ii_v, cat_v, ind_v, ob_v, sem):
        w = lax.axis_index("s") * info.num_cores + lax.axis_index("c")
        base = w * b_per_w
        pltpu.sync_copy(wid_hbm.at[pl.ds(base, b_per_w)], wi_v)
        pltpu.sync_copy(cid_hbm.at[pl.ds(base, b_per_w)], ci_v)
        pltpu.sync_copy(sid_hbm.at[pl.ds(base, b_per_w)], si_v)
        pltpu.sync_copy(iid_hbm.at[pl.ds(base, b_per_w)], ii_v)
        pltpu.sync_copy(wct_hbm, cat_v)
        pltpu.sync_copy(wit_hbm, ind_v)

        iota = lax.iota(jnp.int32, 16)

        @pl.loop(0, b_per_w // 16)
        def _grp(g):
            gbase = g * 16
            wv = wi_v[pl.ds(gbase, 16)]
            cv = ci_v[pl.ds(gbase, 16)]
            sv = si_v[pl.ds(gbase, 16)]
            iv = ii_v[pl.ds(gbase, 16)]
            for j in range(16):
                i = gbase + j
                wid, sid = wv[j], sv[j]
                pltpu.async_copy(ww_hbm.at[wid >> 3, wid & 7],
                                 ob_v.at[i, pl.ds(OFF_W, W_DIM)], sem)
                pltpu.async_copy(ws_hbm.at[sid >> 3, sid & 7],
                                 ob_v.at[i, pl.ds(OFF_S, SC_DIM)], sem)
            row_vec = gbase + iota
            for c in range(C_DIM):
                vvc = plsc.load_gather(
                    cat_v, [jnp.full((16,), c, jnp.int32), cv])
                plsc.store_scatter(
                    ob_v, [row_vec, jnp.full((16,), OFF_C + c, jnp.int32)], vvc)
            for c in range(I_DIM):
                vvi = plsc.load_gather(
                    ind_v, [jnp.full((16,), c, jnp.int32), iv])
                plsc.store_scatter(
                    ob_v, [row_vec, jnp.full((16,), OFF_I + c, jnp.int32)], vvi)

        pltpu.make_async_copy(
            out_hbm.at[pl.ds(0, 384)], ob_v.at[pl.ds(0, 384)], sem).wait()
        pltpu.sync_copy(ob_v, out_hbm.at[pl.ds(base, b_per_w)])

    return sc_gather


_sc_gather = None


def _mlp_body(emb_ref, nwt_ref, npt_ref, nct_ref,
              w1_ref, b1_ref, w2t_ref, b2_ref, w3t_ref, b3_ref, out_ref):
    cdn = (((0,), (0,)), ((), ()))
    x = jnp.dot(emb_ref[...], w1_ref[pl.ds(0, EMB), :],
                preferred_element_type=jnp.float32)
    x += lax.dot_general(nwt_ref[...], w1_ref[pl.ds(EMB, N_W), :], cdn,
                         preferred_element_type=jnp.float32)
    x += lax.dot_general(npt_ref[...], w1_ref[pl.ds(EMB + N_W, N_P), :], cdn,
                         preferred_element_type=jnp.float32)
    x += lax.dot_general(nct_ref[...], w1_ref[pl.ds(EMB + N_W + N_P, N_C), :],
                         cdn, preferred_element_type=jnp.float32)
    x = jnp.maximum(x + b1_ref[...], 0.0)
    x = lax.dot_general(x, w2t_ref[...], (((1,), (1,)), ((), ())),
                        preferred_element_type=jnp.float32)
    x = jnp.maximum(x + b2_ref[...], 0.0)
    out_ref[0, :] = jnp.sum(x * w3t_ref[...], axis=1) + b3_ref[0, 0]


def _mlp(emb, nwt, npt, nct, w1, b1, w2t, b2, w3t, b3, bt=2048):
    grid = B // bt
    ds = lambda i: (i, 0)
    ts = lambda i: (0, i)
    ws = lambda i: (0, 0)
    out = pl.pallas_call(
        _mlp_body,
        grid=(grid,),
        in_specs=[
            pl.BlockSpec((bt, EMB), ds),
            pl.BlockSpec((N_W, bt), ts),
            pl.BlockSpec((N_P, bt), ts),
            pl.BlockSpec((N_C, bt), ts),
            pl.BlockSpec((FC_IN, FC1), ws),
            pl.BlockSpec((1, FC1), ws),
            pl.BlockSpec((FC2, FC1), ws),
            pl.BlockSpec((1, FC2), ws),
            pl.BlockSpec((1, FC2), ws),
            pl.BlockSpec((1, 1), ws),
        ],
        out_specs=pl.BlockSpec((1, bt), ts),
        out_shape=jax.ShapeDtypeStruct((1, B), jnp.float32),
    )(emb, nwt, npt, nct,
      w1, b1.reshape(1, FC1), w2t, b2.reshape(1, FC2),
      w3t, b3.reshape(1, 1))
    return out.T


def kernel(worker_ids, cat_ids, sub_cat_ids, ind_ids,
           numeric_worker_feats, numeric_project_feats, numeric_context_feats,
           W_worker, W_cat, W_sub, W_ind, W1, b1, W2, b2, W3, b3):
    global _sc_gather
    if _sc_gather is None:
        _sc_gather = _make_sc_gather()
    emb = _sc_gather(
        worker_ids.astype(jnp.int32), cat_ids.astype(jnp.int32),
        sub_cat_ids.astype(jnp.int32), ind_ids.astype(jnp.int32),
        W_worker.reshape(NUM_WORKERS // 8, 8, W_DIM),
        W_cat.T,
        W_sub.reshape(NUM_SUBCATS // 8, 8, SC_DIM),
        W_ind.T)
    return _mlp(emb, numeric_worker_feats.T, numeric_project_feats.T,
                numeric_context_feats.T, W1, b1, W2.T, b2, W3.T, b3)


# speedup vs baseline: 1.5204x; 1.0400x over previous
import functools

import jax
import jax.numpy as jnp
from jax import lax
from jax.experimental import pallas as pl
from jax.experimental.pallas import tpu as pltpu
from jax.experimental.pallas import tpu_sc as plsc

B = 16384
W_DIM, C_DIM, SC_DIM, I_DIM = 64, 16, 32, 16
N_W, N_P, N_C = 16, 16, 8
EMB = W_DIM + C_DIM + SC_DIM + I_DIM
FC_IN = EMB + N_W + N_P + N_C
FC1, FC2 = 128, 32
NUM_WORKERS, NUM_CATS, NUM_SUBCATS, NUM_INDS = 1000000, 1000, 100000, 1000

OFF_W, OFF_C, OFF_S, OFF_I = 0, W_DIM, W_DIM + C_DIM, W_DIM + C_DIM + SC_DIM


def _make_sc_gather():
    info = plsc.get_sparse_core_info()
    nw = info.num_cores * info.num_subcores
    b_per_w = B // nw
    mesh = plsc.VectorSubcoreMesh(core_axis_name="c", subcore_axis_name="s")

    @functools.partial(
        pl.kernel,
        mesh=mesh,
        compiler_params=pltpu.CompilerParams(needs_layout_passes=False),
        out_type=jax.ShapeDtypeStruct((B, EMB), jnp.float32),
        scratch_types=[
            pltpu.VMEM((b_per_w,), jnp.int32),
            pltpu.VMEM((b_per_w,), jnp.int32),
            pltpu.VMEM((b_per_w,), jnp.int32),
            pltpu.VMEM((b_per_w,), jnp.int32),
            pltpu.VMEM((b_per_w, EMB), jnp.float32),
            pltpu.SemaphoreType.DMA,
        ],
    )
    def sc_gather(wid_hbm, cid_hbm, sid_hbm, iid_hbm,
                  ww_hbm, wc_hbm, ws_hbm, wi_hbm2,
                  out_hbm,
                  wi_v, ci_v, si_v, ii_v, ob_v, sem):
        w = lax.axis_index("s") * info.num_cores + lax.axis_index("c")
        base = w * b_per_w
        pltpu.sync_copy(wid_hbm.at[pl.ds(base, b_per_w)], wi_v)
        pltpu.sync_copy(cid_hbm.at[pl.ds(base, b_per_w)], ci_v)
        pltpu.sync_copy(sid_hbm.at[pl.ds(base, b_per_w)], si_v)
        pltpu.sync_copy(iid_hbm.at[pl.ds(base, b_per_w)], ii_v)
        @pl.loop(0, b_per_w // 16)
        def _grp(g):
            gbase = g * 16
            wv = wi_v[pl.ds(gbase, 16)]
            cv = ci_v[pl.ds(gbase, 16)]
            sv = si_v[pl.ds(gbase, 16)]
            iv = ii_v[pl.ds(gbase, 16)]
            for j in range(16):
                i = gbase + j
                wid, cid, sid, iid = wv[j], cv[j], sv[j], iv[j]
                pltpu.async_copy(ww_hbm.at[wid >> 3, wid & 7],
                                 ob_v.at[i, pl.ds(OFF_W, W_DIM)], sem)
                pltpu.async_copy(wc_hbm.at[cid >> 3, cid & 7],
                                 ob_v.at[i, pl.ds(OFF_C, C_DIM)], sem)
                pltpu.async_copy(ws_hbm.at[sid >> 3, sid & 7],
                                 ob_v.at[i, pl.ds(OFF_S, SC_DIM)], sem)
                pltpu.async_copy(wi_hbm2.at[iid >> 3, iid & 7],
                                 ob_v.at[i, pl.ds(OFF_I, I_DIM)], sem)

        pltpu.make_async_copy(
            out_hbm.at[pl.ds(0, b_per_w)], ob_v, sem).wait()
        pltpu.sync_copy(ob_v, out_hbm.at[pl.ds(base, b_per_w)])

    return sc_gather


_sc_gather = None


def _mlp_body(emb_ref, nwt_ref, npt_ref, nct_ref,
              w1_ref, b1_ref, w2t_ref, b2_ref, w3t_ref, b3_ref, out_ref):
    cdn = (((0,), (0,)), ((), ()))
    x = jnp.dot(emb_ref[...], w1_ref[pl.ds(0, EMB), :],
                preferred_element_type=jnp.float32)
    x += lax.dot_general(nwt_ref[...], w1_ref[pl.ds(EMB, N_W), :], cdn,
                         preferred_element_type=jnp.float32)
    x += lax.dot_general(npt_ref[...], w1_ref[pl.ds(EMB + N_W, N_P), :], cdn,
                         preferred_element_type=jnp.float32)
    x += lax.dot_general(nct_ref[...], w1_ref[pl.ds(EMB + N_W + N_P, N_C), :],
                         cdn, preferred_element_type=jnp.float32)
    x = jnp.maximum(x + b1_ref[...], 0.0)
    x = lax.dot_general(x, w2t_ref[...], (((1,), (1,)), ((), ())),
                        preferred_element_type=jnp.float32)
    x = jnp.maximum(x + b2_ref[...], 0.0)
    out_ref[0, :] = jnp.sum(x * w3t_ref[...], axis=1) + b3_ref[0, 0]


def _mlp(emb, nwt, npt, nct, w1, b1, w2t, b2, w3t, b3, bt=2048):
    grid = B // bt
    ds = lambda i: (i, 0)
    ts = lambda i: (0, i)
    ws = lambda i: (0, 0)
    out = pl.pallas_call(
        _mlp_body,
        grid=(grid,),
        in_specs=[
            pl.BlockSpec((bt, EMB), ds),
            pl.BlockSpec((N_W, bt), ts),
            pl.BlockSpec((N_P, bt), ts),
            pl.BlockSpec((N_C, bt), ts),
            pl.BlockSpec((FC_IN, FC1), ws),
            pl.BlockSpec((1, FC1), ws),
            pl.BlockSpec((FC2, FC1), ws),
            pl.BlockSpec((1, FC2), ws),
            pl.BlockSpec((1, FC2), ws),
            pl.BlockSpec((1, 1), ws),
        ],
        out_specs=pl.BlockSpec((1, bt), ts),
        out_shape=jax.ShapeDtypeStruct((1, B), jnp.float32),
    )(emb, nwt, npt, nct,
      w1, b1.reshape(1, FC1), w2t, b2.reshape(1, FC2),
      w3t, b3.reshape(1, 1))
    return out.T


def kernel(worker_ids, cat_ids, sub_cat_ids, ind_ids,
           numeric_worker_feats, numeric_project_feats, numeric_context_feats,
           W_worker, W_cat, W_sub, W_ind, W1, b1, W2, b2, W3, b3):
    global _sc_gather
    if _sc_gather is None:
        _sc_gather = _make_sc_gather()
    emb = _sc_gather(
        worker_ids.astype(jnp.int32), cat_ids.astype(jnp.int32),
        sub_cat_ids.astype(jnp.int32), ind_ids.astype(jnp.int32),
        W_worker.reshape(NUM_WORKERS // 8, 8, W_DIM),
        W_cat.reshape(NUM_CATS // 8, 8, C_DIM),
        W_sub.reshape(NUM_SUBCATS // 8, 8, SC_DIM),
        W_ind.reshape(NUM_INDS // 8, 8, I_DIM))
    return _mlp(emb, numeric_worker_feats.T, numeric_project_feats.T,
                numeric_context_feats.T, W1, b1, W2.T, b2, W3.T, b3)


# speedup vs baseline: 1.5292x; 1.0058x over previous
import functools

import jax
import jax.numpy as jnp
from jax import lax
from jax.experimental import pallas as pl
from jax.experimental.pallas import tpu as pltpu
from jax.experimental.pallas import tpu_sc as plsc

B = 16384
W_DIM, C_DIM, SC_DIM, I_DIM = 64, 16, 32, 16
N_W, N_P, N_C = 16, 16, 8
EMB = W_DIM + C_DIM + SC_DIM + I_DIM
FC_IN = EMB + N_W + N_P + N_C
FC1, FC2 = 128, 32
NUM_WORKERS, NUM_CATS, NUM_SUBCATS, NUM_INDS = 1000000, 1000, 100000, 1000

OFF_W, OFF_C, OFF_S, OFF_I = 0, W_DIM, W_DIM + C_DIM, W_DIM + C_DIM + SC_DIM


def _make_sc_gather():
    info = plsc.get_sparse_core_info()
    nw = info.num_cores * info.num_subcores
    b_per_w = B // nw
    mesh = plsc.VectorSubcoreMesh(core_axis_name="c", subcore_axis_name="s")

    @functools.partial(
        pl.kernel,
        mesh=mesh,
        compiler_params=pltpu.CompilerParams(needs_layout_passes=False),
        out_type=jax.ShapeDtypeStruct((B, EMB), jnp.float32),
        scratch_types=[
            pltpu.VMEM((b_per_w,), jnp.int32),
            pltpu.VMEM((b_per_w,), jnp.int32),
            pltpu.VMEM((b_per_w,), jnp.int32),
            pltpu.VMEM((b_per_w,), jnp.int32),
            pltpu.VMEM((b_per_w, EMB), jnp.float32),
            pltpu.SemaphoreType.DMA,
        ],
    )
    def sc_gather(wid_hbm, cid_hbm, sid_hbm, iid_hbm,
                  ww_hbm, wc_hbm, ws_hbm, wi_hbm2,
                  out_hbm,
                  wi_v, ci_v, si_v, ii_v, ob_v, sem):
        w = lax.axis_index("s") * info.num_cores + lax.axis_index("c")
        base = w * b_per_w
        pltpu.sync_copy(wid_hbm.at[pl.ds(base, b_per_w)], wi_v)
        pltpu.sync_copy(cid_hbm.at[pl.ds(base, b_per_w)], ci_v)
        pltpu.sync_copy(sid_hbm.at[pl.ds(base, b_per_w)], si_v)
        pltpu.sync_copy(iid_hbm.at[pl.ds(base, b_per_w)], ii_v)
        @pl.loop(0, b_per_w // 16)
        def _grp(g):
            gbase = g * 16
            wv = wi_v[pl.ds(gbase, 16)]
            cv = ci_v[pl.ds(gbase, 16)]
            sv = si_v[pl.ds(gbase, 16)]
            iv = ii_v[pl.ds(gbase, 16)]
            for j in range(16):
                i = gbase + j
                wid, cid, sid, iid = wv[j], cv[j], sv[j], iv[j]
                pltpu.async_copy(ww_hbm.at[wid >> 3, wid & 7],
                                 ob_v.at[i, pl.ds(OFF_W, W_DIM)], sem)
                pltpu.async_copy(wc_hbm.at[cid >> 3, cid & 7],
                                 ob_v.at[i, pl.ds(OFF_C, C_DIM)], sem)
                pltpu.async_copy(ws_hbm.at[sid >> 3, sid & 7],
                                 ob_v.at[i, pl.ds(OFF_S, SC_DIM)], sem)
                pltpu.async_copy(wi_hbm2.at[iid >> 3, iid & 7],
                                 ob_v.at[i, pl.ds(OFF_I, I_DIM)], sem)

        pltpu.make_async_copy(
            out_hbm.at[pl.ds(0, b_per_w)], ob_v, sem).wait()
        pltpu.sync_copy(ob_v, out_hbm.at[pl.ds(base, b_per_w)])

    return sc_gather


_sc_gather = None


def _mlp_body(emb_ref, nwt_ref, npt_ref, nct_ref,
              w1_ref, b1_ref, w2t_ref, b2_ref, w3t_ref, b3_ref, out_ref):
    cdn = (((0,), (0,)), ((), ()))
    x = jnp.dot(emb_ref[...], w1_ref[pl.ds(0, EMB), :],
                preferred_element_type=jnp.float32)
    x += lax.dot_general(nwt_ref[...], w1_ref[pl.ds(EMB, N_W), :], cdn,
                         preferred_element_type=jnp.float32)
    x += lax.dot_general(npt_ref[...], w1_ref[pl.ds(EMB + N_W, N_P), :], cdn,
                         preferred_element_type=jnp.float32)
    x += lax.dot_general(nct_ref[...], w1_ref[pl.ds(EMB + N_W + N_P, N_C), :],
                         cdn, preferred_element_type=jnp.float32)
    x = jnp.maximum(x + b1_ref[...], 0.0)
    x = lax.dot_general(x, w2t_ref[...], (((1,), (1,)), ((), ())),
                        preferred_element_type=jnp.float32)
    x = jnp.maximum(x + b2_ref[...], 0.0)
    out_ref[0, :] = jnp.sum(x * w3t_ref[...], axis=1) + b3_ref[0, 0]


def _mlp(emb, nwt, npt, nct, w1, b1, w2t, b2, w3t, b3, bt=4096):
    grid = B // bt
    ds = lambda i: (i, 0)
    ts = lambda i: (0, i)
    ws = lambda i: (0, 0)
    out = pl.pallas_call(
        _mlp_body,
        grid=(grid,),
        in_specs=[
            pl.BlockSpec((bt, EMB), ds),
            pl.BlockSpec((N_W, bt), ts),
            pl.BlockSpec((N_P, bt), ts),
            pl.BlockSpec((N_C, bt), ts),
            pl.BlockSpec((FC_IN, FC1), ws),
            pl.BlockSpec((1, FC1), ws),
            pl.BlockSpec((FC2, FC1), ws),
            pl.BlockSpec((1, FC2), ws),
            pl.BlockSpec((1, FC2), ws),
            pl.BlockSpec((1, 1), ws),
        ],
        out_specs=pl.BlockSpec((1, bt), ts),
        out_shape=jax.ShapeDtypeStruct((1, B), jnp.float32),
    )(emb, nwt, npt, nct,
      w1, b1.reshape(1, FC1), w2t, b2.reshape(1, FC2),
      w3t, b3.reshape(1, 1))
    return out.T


def kernel(worker_ids, cat_ids, sub_cat_ids, ind_ids,
           numeric_worker_feats, numeric_project_feats, numeric_context_feats,
           W_worker, W_cat, W_sub, W_ind, W1, b1, W2, b2, W3, b3):
    global _sc_gather
    if _sc_gather is None:
        _sc_gather = _make_sc_gather()
    emb = _sc_gather(
        worker_ids.astype(jnp.int32), cat_ids.astype(jnp.int32),
        sub_cat_ids.astype(jnp.int32), ind_ids.astype(jnp.int32),
        W_worker.reshape(NUM_WORKERS // 8, 8, W_DIM),
        W_cat.reshape(NUM_CATS // 8, 8, C_DIM),
        W_sub.reshape(NUM_SUBCATS // 8, 8, SC_DIM),
        W_ind.reshape(NUM_INDS // 8, 8, I_DIM))
    return _mlp(emb, numeric_worker_feats.T, numeric_project_feats.T,
                numeric_context_feats.T, W1, b1, W2.T, b2, W3.T, b3)
